# baseline (speedup 1.0000x reference)
import functools

import jax
import jax.numpy as jnp
from jax.experimental import pallas as pl
from jax.experimental.pallas import tpu as pltpu

_K = 8
_BB = 16
_NBUF = 8
_AHEAD = _NBUF - 1


def _sample_concrete_block(logits_ref, uniform_hbm, out_ref, ubuf, ebuf, sems,
                           *, nsteps):
    nb = pl.program_id(0)

    def start_dma(step, slot):
        b = step // _K
        kk = step % _K
        pltpu.make_async_copy(
            uniform_hbm.at[pl.ds(b * _BB, _BB), kk],
            ubuf.at[slot],
            sems.at[slot],
        ).start()

    @pl.when(nb == 0)
    def _prologue():
        for i in range(_AHEAD):
            start_dma(i, i % _NBUF)

    L = logits_ref[:]
    Lmax = jnp.max(L, axis=-1, keepdims=True)
    expL = jnp.exp(2.0 * (L - Lmax))
    eps = jnp.finfo(jnp.float32).eps

    rs = []
    for k in range(_K):
        g = nb * _K + k

        @pl.when(g + _AHEAD < nsteps)
        def _next_dma():
            start_dma(g + _AHEAD, (k + _AHEAD) % _NBUF)

        pltpu.make_async_copy(
            uniform_hbm.at[pl.ds(0, _BB), 0],
            ubuf.at[k],
            sems.at[k],
        ).wait()
        u = jnp.clip(ubuf[k], eps, 1.0)
        rw = pl.reciprocal(jnp.log2(u), approx=True)
        e = expL * (rw * rw)
        ebuf[k] = e
        s = jnp.sum(e, axis=-1, keepdims=True)
        rs.append(pl.reciprocal(s, approx=True))

    v = [ebuf[k] * rs[k] for k in range(_K)]
    while len(v) > 1:
        v = [jnp.maximum(v[i], v[i + 1]) for i in range(0, len(v), 2)]
    out_ref[:] = v[0]


@functools.partial(jax.jit, static_argnames=("interpret",))
def kernel(logits, uniform, interpret=False):
    B, D = logits.shape
    _, K, _ = uniform.shape
    nblocks = B // _BB
    nsteps = nblocks * _K
    return pl.pallas_call(
        functools.partial(_sample_concrete_block, nsteps=nsteps),
        grid=(nblocks,),
        in_specs=[
            pl.BlockSpec((_BB, D), lambda b: (b, 0)),
            pl.BlockSpec(memory_space=pl.ANY),
        ],
        out_specs=pl.BlockSpec((_BB, D), lambda b: (b, 0)),
        out_shape=jax.ShapeDtypeStruct((B, D), jnp.float32),
        scratch_shapes=[
            pltpu.VMEM((_NBUF, _BB, D), jnp.float32),
            pltpu.VMEM((_K, _BB, D), jnp.float32),
            pltpu.SemaphoreType.DMA((_NBUF,)),
        ],
        interpret=interpret,
    )(logits, uniform)


# speedup vs baseline: 1.3953x; 1.3953x over previous
import functools

import jax
import jax.numpy as jnp
from jax.experimental import pallas as pl
from jax.experimental.pallas import tpu as pltpu

_K = 8
_BB = 32
_NBUF = 4
_AHEAD = _NBUF - 1


def _sample_concrete_block(logits_ref, uniform_hbm, out_ref, ubuf, sems, *,
                           nsteps):
    nb = pl.program_id(0)

    def start_dma(step, slot):
        b = step // _K
        kk = step % _K
        pltpu.make_async_copy(
            uniform_hbm.at[pl.ds(b * _BB, _BB), kk],
            ubuf.at[slot],
            sems.at[slot],
        ).start()

    @pl.when(nb == 0)
    def _prologue():
        for i in range(_AHEAD):
            start_dma(i, i % _NBUF)

    L = logits_ref[:]
    Lmax = jnp.max(L, axis=-1, keepdims=True)
    expL = jnp.exp(2.0 * (L - Lmax))
    eps = jnp.finfo(jnp.float32).eps

    acc = None
    for k in range(_K):
        g = nb * _K + k

        @pl.when(g + _AHEAD < nsteps)
        def _next_dma():
            start_dma(g + _AHEAD, (k + _AHEAD) % _NBUF)

        pltpu.make_async_copy(
            uniform_hbm.at[pl.ds(0, _BB), 0],
            ubuf.at[k % _NBUF],
            sems.at[k % _NBUF],
        ).wait()
        u = jnp.clip(ubuf[k % _NBUF], eps, 1.0)
        rw = pl.reciprocal(jnp.log2(u), approx=True)
        e = expL * (rw * rw)
        s = jnp.sum(e, axis=-1, keepdims=True)
        v = e * pl.reciprocal(s, approx=True)
        acc = v if k == 0 else jnp.maximum(acc, v)

    out_ref[:] = acc


@functools.partial(jax.jit, static_argnames=("interpret",))
def kernel(logits, uniform, interpret=False):
    B, D = logits.shape
    _, K, _ = uniform.shape
    nblocks = B // _BB
    nsteps = nblocks * _K
    return pl.pallas_call(
        functools.partial(_sample_concrete_block, nsteps=nsteps),
        grid=(nblocks,),
        in_specs=[
            pl.BlockSpec((_BB, D), lambda b: (b, 0)),
            pl.BlockSpec(memory_space=pl.ANY),
        ],
        out_specs=pl.BlockSpec((_BB, D), lambda b: (b, 0)),
        out_shape=jax.ShapeDtypeStruct((B, D), jnp.float32),
        scratch_shapes=[
            pltpu.VMEM((_NBUF, _BB, D), jnp.float32),
            pltpu.SemaphoreType.DMA((_NBUF,)),
        ],
        interpret=interpret,
    )(logits, uniform)
